# Initial kernel scaffold; baseline (speedup 1.0000x reference)
import jax
import jax.numpy as jnp
from jax.experimental import pallas as pl


def kernel(x, edge_index, W1, b1, W2, b2, Wp, bp):
    raise NotImplementedError("write your pallas kernel here")


# speedup vs baseline: 21.1514x; 21.1514x over previous
import functools

import jax
import jax.numpy as jnp
from jax import lax
from jax.experimental import pallas as pl
from jax.experimental.pallas import tpu as pltpu
from jax.experimental.pallas import tpu_sc as plsc

N = 10000
NP = 10240
E = 320000
EP = 327680
EB = EP // 128
D = 128
DH = 64
OUT = 64
NC = 2
NS = 16
RPT = NP // NS
BPT = EB // NS
DEG_BPT = EB // (NC * NS)
CB = 4
ZR = 320

_SC_MESH = dict(core_axis_name="c", subcore_axis_name="s",
                num_cores=NC, num_subcores=NS)


def _zero_rows(ref, nrows, ncol16):
    z = jnp.zeros((16,), jnp.float32)

    def body(i, _):
        for c in range(ncol16):
            ref[i, pl.ds(c * 16, 16)] = z
        return 0

    lax.fori_loop(0, nrows, body, 0)


@functools.partial(
    pl.kernel,
    out_type=jax.ShapeDtypeStruct((NC, NP), jnp.float32),
    mesh=plsc.VectorSubcoreMesh(**_SC_MESH),
    scratch_types=[
        pltpu.VMEM((DEG_BPT, 128), jnp.int32),
        pltpu.VMEM((128,), jnp.float32),
        pltpu.VMEM((RPT,), jnp.float32),
        pltpu.VMEM_SHARED((NP,), jnp.float32),
    ],
)
def _sc_degree(ei_hbm, deg_out, dst_buf, ones_buf, row_buf, deg_shared):
    c = lax.axis_index("c")
    s = lax.axis_index("s")
    ones16 = jnp.ones((16,), jnp.float32)
    zeros16 = jnp.zeros((16,), jnp.float32)
    for i in range(8):
        ones_buf[pl.ds(i * 16, 16)] = ones16

    def zbody(i, _):
        row_buf[pl.ds(i * 16, 16)] = zeros16
        return 0
    lax.fori_loop(0, RPT // 16, zbody, 0)
    pltpu.sync_copy(row_buf, deg_shared.at[pl.ds(s * RPT, RPT)])

    b0 = c * (EB // NC) + s * DEG_BPT
    pltpu.sync_copy(ei_hbm.at[1, pl.ds(b0, DEG_BPT), :], dst_buf)
    plsc.subcore_barrier()

    def body(k, _):
        pltpu.sync_copy(ones_buf, deg_shared.at[dst_buf.at[k]], add=True)
        return 0
    lax.fori_loop(0, DEG_BPT, body, 0)

    plsc.subcore_barrier()
    pltpu.sync_copy(deg_shared.at[pl.ds(s * RPT, RPT)], row_buf)
    pltpu.sync_copy(row_buf, deg_out.at[c, pl.ds(s * RPT, RPT)])


@functools.partial(
    pl.kernel,
    out_type=jax.ShapeDtypeStruct((NC, NP, DH), jnp.float32),
    mesh=plsc.VectorSubcoreMesh(**_SC_MESH),
    scratch_types=[
        pltpu.VMEM((CB, 128), jnp.int32),
        pltpu.VMEM((CB, 128), jnp.int32),
        pltpu.VMEM((CB * 128, DH), jnp.float32),
        pltpu.VMEM_SHARED((NP, DH), jnp.float32),
        pltpu.VMEM_SHARED((NP, DH), jnp.float32),
        pltpu.SemaphoreType.DMA,
    ],
    compiler_params=pltpu.CompilerParams(use_tc_tiling_on_sc=False),
)
def _sc_prop(g_hbm, ei_hbm, acc_out, src_buf, dst_buf, rows,
             g_shared, acc_shared, sem):
    c = lax.axis_index("c")
    s = lax.axis_index("s")

    pltpu.sync_copy(g_hbm.at[c, pl.ds(s * RPT, RPT), :],
                    g_shared.at[pl.ds(s * RPT, RPT), :])
    _zero_rows(rows, ZR, DH // 16)
    for z in range(RPT // ZR):
        pltpu.sync_copy(rows.at[pl.ds(0, ZR), :],
                        acc_shared.at[pl.ds(s * RPT + z * ZR, ZR), :])
    plsc.subcore_barrier()

    def chunk(k, _):
        b0 = s * BPT + k * CB
        pltpu.sync_copy(ei_hbm.at[0, pl.ds(b0, CB), :], src_buf)
        pltpu.sync_copy(ei_hbm.at[1, pl.ds(b0, CB), :], dst_buf)
        descs = []
        for j in range(CB):
            descs.append(pltpu.async_copy(
                g_shared.at[src_buf.at[j]],
                rows.at[pl.ds(j * 128, 128), :], sem))
        for d in descs:
            d.wait()
        for j in range(CB):
            pltpu.sync_copy(rows.at[pl.ds(j * 128, 128), :],
                            acc_shared.at[dst_buf.at[j]], add=True)
        return 0

    lax.fori_loop(0, BPT // CB, chunk, 0)

    plsc.subcore_barrier()
    pltpu.sync_copy(acc_shared.at[pl.ds(s * RPT, RPT), :],
                    acc_out.at[c, pl.ds(s * RPT, RPT), :])


BM = 1024
_GRID = NP // BM


def _dinv_col(degb):
    ones = jnp.ones((NC, 1), jnp.float32)
    deg = lax.dot_general(degb, ones, (((0,), (0,)), ((), ())),
                          preferred_element_type=jnp.float32)
    return lax.rsqrt(deg + 1.0)


def _tc1_body(x_ref, w1_ref, deg_ref, g_ref):
    dinv = _dinv_col(deg_ref[...])
    hw = jnp.dot(x_ref[...], w1_ref[...], preferred_element_type=jnp.float32)
    g = hw * dinv
    g_ref[0] = g[:, :DH]
    g_ref[1] = g[:, DH:]


def _tc2_body(acc_ref, g_ref, deg_ref, b1_ref, w2_ref, g2_ref):
    dinv = _dinv_col(deg_ref[...])
    b1 = b1_ref[...]
    h_lo = jnp.maximum(dinv * (acc_ref[0] + g_ref[0]) + b1[:, :DH], 0.0)
    h_hi = jnp.maximum(dinv * (acc_ref[1] + g_ref[1]) + b1[:, DH:], 0.0)
    w2 = w2_ref[...]
    hw = (jnp.dot(h_lo, w2[:DH, :], preferred_element_type=jnp.float32)
          + jnp.dot(h_hi, w2[DH:, :], preferred_element_type=jnp.float32))
    g2 = hw * dinv
    g2_ref[0] = g2[:, :DH]
    g2_ref[1] = g2[:, DH:]


def _tc3_body(acc_ref, g_ref, deg_ref, b2_ref, wp_ref, bp_ref, z_ref):
    dinv = _dinv_col(deg_ref[...])
    b2 = b2_ref[...]
    h_lo = jnp.maximum(dinv * (acc_ref[0] + g_ref[0]) + b2[:, :DH], 0.0)
    h_hi = jnp.maximum(dinv * (acc_ref[1] + g_ref[1]) + b2[:, DH:], 0.0)
    wp = wp_ref[...]
    z_ref[...] = (jnp.dot(h_lo, wp[:DH, :], preferred_element_type=jnp.float32)
                  + jnp.dot(h_hi, wp[DH:, :], preferred_element_type=jnp.float32)
                  + bp_ref[...])


def _spec_halves():
    return pl.BlockSpec((NC, BM, DH), lambda i: (0, i, 0))


def _spec_deg():
    return pl.BlockSpec((NC, BM), lambda i: (0, i))


def _spec_full(r, c):
    return pl.BlockSpec((r, c), lambda i: (0, 0))


_tc1 = pl.pallas_call(
    _tc1_body,
    grid=(_GRID,),
    in_specs=[pl.BlockSpec((BM, D), lambda i: (i, 0)),
              _spec_full(D, D),
              _spec_deg()],
    out_specs=_spec_halves(),
    out_shape=jax.ShapeDtypeStruct((NC, NP, DH), jnp.float32),
)

_tc2 = pl.pallas_call(
    _tc2_body,
    grid=(_GRID,),
    in_specs=[_spec_halves(), _spec_halves(), _spec_deg(),
              _spec_full(1, D), _spec_full(D, D)],
    out_specs=_spec_halves(),
    out_shape=jax.ShapeDtypeStruct((NC, NP, DH), jnp.float32),
)

_tc3 = pl.pallas_call(
    _tc3_body,
    grid=(_GRID,),
    in_specs=[_spec_halves(), _spec_halves(), _spec_deg(),
              _spec_full(1, D), _spec_full(D, OUT), _spec_full(1, OUT)],
    out_specs=pl.BlockSpec((BM, OUT), lambda i: (i, 0)),
    out_shape=jax.ShapeDtypeStruct((NP, OUT), jnp.float32),
)


def kernel(x, edge_index, W1, b1, W2, b2, Wp, bp):
    ei = edge_index.astype(jnp.int32)
    pad_idx = N + (jnp.arange(EP - E, dtype=jnp.int32) % (NP - N))
    ei_p = jnp.concatenate(
        [ei, jnp.stack([pad_idx, pad_idx])], axis=1).reshape(2, EB, 128)
    x_p = jnp.pad(x, ((0, NP - N), (0, 0)))

    deg_parts = _sc_degree(ei_p)
    g1 = _tc1(x_p, W1, deg_parts)
    acc1 = _sc_prop(g1, ei_p)
    g2 = _tc2(acc1, g1, deg_parts, b1.reshape(1, D), W2)
    acc2 = _sc_prop(g2, ei_p)
    z = _tc3(acc2, g2, deg_parts, b2.reshape(1, D), Wp, bp.reshape(1, OUT))
    return z[:N]


# speedup vs baseline: 34.2609x; 1.6198x over previous
import functools

import jax
import jax.numpy as jnp
from jax import lax
from jax.experimental import pallas as pl
from jax.experimental.pallas import tpu as pltpu
from jax.experimental.pallas import tpu_sc as plsc

N = 10000
NP = 10240
E = 320000
EP = 327680
EB = EP // 128
D = 128
OUT = 64
NC = 2
NS = 16
RPT = NP // NS
BPT = EB // (NC * NS)
HB = BPT // 2

_SC_MESH = dict(core_axis_name="c", subcore_axis_name="s",
                num_cores=NC, num_subcores=NS)


@functools.partial(
    pl.kernel,
    out_type=jax.ShapeDtypeStruct((NC, NP), jnp.float32),
    mesh=plsc.VectorSubcoreMesh(**_SC_MESH),
    scratch_types=[
        pltpu.VMEM((BPT, 128), jnp.int32),
        pltpu.VMEM((128,), jnp.float32),
        pltpu.VMEM((RPT,), jnp.float32),
        pltpu.VMEM_SHARED((NP,), jnp.float32),
    ],
)
def _sc_degree(ei_hbm, deg_out, dst_buf, ones_buf, row_buf, deg_shared):
    c = lax.axis_index("c")
    s = lax.axis_index("s")
    ones16 = jnp.ones((16,), jnp.float32)
    zeros16 = jnp.zeros((16,), jnp.float32)
    for i in range(8):
        ones_buf[pl.ds(i * 16, 16)] = ones16

    def zbody(i, _):
        row_buf[pl.ds(i * 16, 16)] = zeros16
        return 0
    lax.fori_loop(0, RPT // 16, zbody, 0)
    pltpu.sync_copy(row_buf, deg_shared.at[pl.ds(s * RPT, RPT)])

    b0 = c * (EB // NC) + s * BPT
    pltpu.sync_copy(ei_hbm.at[1, pl.ds(b0, BPT), :], dst_buf)
    plsc.subcore_barrier()

    def body(k, _):
        pltpu.sync_copy(ones_buf, deg_shared.at[dst_buf.at[k]], add=True)
        return 0
    lax.fori_loop(0, BPT, body, 0)

    plsc.subcore_barrier()
    pltpu.sync_copy(deg_shared.at[pl.ds(s * RPT, RPT)], row_buf)
    pltpu.sync_copy(row_buf, deg_out.at[c, pl.ds(s * RPT, RPT)])


@functools.partial(
    pl.kernel,
    out_type=jax.ShapeDtypeStruct((NC, NP, D), jnp.float32),
    mesh=plsc.VectorSubcoreMesh(**_SC_MESH),
    scratch_types=[
        pltpu.VMEM((2, HB, 128), jnp.int32),
        pltpu.VMEM((128, D), jnp.float32),
        pltpu.VMEM((128, D), jnp.float32),
        pltpu.VMEM_SHARED((NP, D), jnp.float32),
        pltpu.SemaphoreType.DMA,
        pltpu.SemaphoreType.DMA,
    ],
    compiler_params=pltpu.CompilerParams(use_tc_tiling_on_sc=False),
)
def _sc_prop(g_hbm, ei_hbm, acc_out, idx_buf, rows_a, rows_b,
             acc_shared, sem_a, sem_b):
    c = lax.axis_index("c")
    s = lax.axis_index("s")

    zeros16 = jnp.zeros((16,), jnp.float32)

    def zbody(i, _):
        for j in range(D // 16):
            rows_a[i, pl.ds(j * 16, 16)] = zeros16
        return 0
    lax.fori_loop(0, 128, zbody, 0)
    for k in range(RPT // 128):
        pltpu.sync_copy(rows_a, acc_shared.at[pl.ds(s * RPT + k * 128, 128), :])
    plsc.subcore_barrier()

    b0 = c * (EB // NC) + s * BPT

    def _gwait(sem):
        pltpu.make_async_copy(g_hbm.at[pl.ds(0, 128), :], rows_a, sem).wait()

    for half in range(2):
        pltpu.sync_copy(ei_hbm.at[:, pl.ds(b0 + half * HB, HB), :], idx_buf)
        pltpu.async_copy(g_hbm.at[idx_buf.at[0, 0]], rows_a, sem_a)

        def step(t, _):
            e = 2 * t
            pltpu.async_copy(g_hbm.at[idx_buf.at[0, e + 1]], rows_b, sem_b)
            _gwait(sem_a)
            pltpu.sync_copy(rows_a, acc_shared.at[idx_buf.at[1, e]],
                            add=True)

            @pl.when(e + 2 < HB)
            def _():
                pltpu.async_copy(g_hbm.at[idx_buf.at[0, e + 2]], rows_a, sem_a)

            _gwait(sem_b)
            pltpu.sync_copy(rows_b, acc_shared.at[idx_buf.at[1, e + 1]],
                            add=True)
            return 0

        lax.fori_loop(0, HB // 2, step, 0)

    plsc.subcore_barrier()
    for k in range(RPT // 128):
        sl = pl.ds(s * RPT + k * 128, 128)
        pltpu.sync_copy(acc_shared.at[sl, :], acc_out.at[c, sl, :])


BM = 1024
_GRID = NP // BM


def _dinv_col(degb):
    ones = jnp.ones((NC, 1), jnp.float32)
    deg = lax.dot_general(degb, ones, (((0,), (0,)), ((), ())),
                          preferred_element_type=jnp.float32)
    return lax.rsqrt(deg + 1.0)


def _tc1_body(x_ref, w1_ref, deg_ref, g_ref):
    dinv = _dinv_col(deg_ref[...])
    hw = jnp.dot(x_ref[...], w1_ref[...], preferred_element_type=jnp.float32)
    g_ref[...] = hw * dinv


def _tc2_body(acc_ref, g_ref, deg_ref, b1_ref, w2_ref, g2_ref):
    dinv = _dinv_col(deg_ref[...])
    h = jnp.maximum(dinv * (acc_ref[0] + acc_ref[1] + g_ref[...])
                    + b1_ref[...], 0.0)
    g2_ref[...] = jnp.dot(h, w2_ref[...],
                          preferred_element_type=jnp.float32) * dinv


def _tc3_body(acc_ref, g_ref, deg_ref, b2_ref, wp_ref, bp_ref, z_ref):
    dinv = _dinv_col(deg_ref[...])
    h = jnp.maximum(dinv * (acc_ref[0] + acc_ref[1] + g_ref[...])
                    + b2_ref[...], 0.0)
    z_ref[...] = jnp.dot(h, wp_ref[...],
                         preferred_element_type=jnp.float32) + bp_ref[...]


def _spec_rows(cols):
    return pl.BlockSpec((BM, cols), lambda i: (i, 0))


def _spec_acc():
    return pl.BlockSpec((NC, BM, D), lambda i: (0, i, 0))


def _spec_deg():
    return pl.BlockSpec((NC, BM), lambda i: (0, i))


def _spec_full(r, c):
    return pl.BlockSpec((r, c), lambda i: (0, 0))


_tc1 = pl.pallas_call(
    _tc1_body,
    grid=(_GRID,),
    in_specs=[_spec_rows(D), _spec_full(D, D), _spec_deg()],
    out_specs=_spec_rows(D),
    out_shape=jax.ShapeDtypeStruct((NP, D), jnp.float32),
)

_tc2 = pl.pallas_call(
    _tc2_body,
    grid=(_GRID,),
    in_specs=[_spec_acc(), _spec_rows(D), _spec_deg(),
              _spec_full(1, D), _spec_full(D, D)],
    out_specs=_spec_rows(D),
    out_shape=jax.ShapeDtypeStruct((NP, D), jnp.float32),
)

_tc3 = pl.pallas_call(
    _tc3_body,
    grid=(_GRID,),
    in_specs=[_spec_acc(), _spec_rows(D), _spec_deg(),
              _spec_full(1, D), _spec_full(D, OUT), _spec_full(1, OUT)],
    out_specs=pl.BlockSpec((BM, OUT), lambda i: (i, 0)),
    out_shape=jax.ShapeDtypeStruct((NP, OUT), jnp.float32),
)


def kernel(x, edge_index, W1, b1, W2, b2, Wp, bp):
    ei = edge_index.astype(jnp.int32)
    pad_idx = N + (jnp.arange(EP - E, dtype=jnp.int32) % (NP - N))
    ei_p = jnp.concatenate(
        [ei, jnp.stack([pad_idx, pad_idx])], axis=1).reshape(2, EB, 128)
    x_p = jnp.pad(x, ((0, NP - N), (0, 0)))

    deg_parts = _sc_degree(ei_p)
    g1 = _tc1(x_p, W1, deg_parts)
    acc1 = _sc_prop(g1, ei_p)
    g2 = _tc2(acc1, g1, deg_parts, b1.reshape(1, D), W2)
    acc2 = _sc_prop(g2, ei_p)
    z = _tc3(acc2, g2, deg_parts, b2.reshape(1, D), Wp, bp.reshape(1, OUT))
    return z[:N]
